# baseline (speedup 1.0000x reference)
import functools

import jax
import jax.numpy as jnp
from jax import lax
from jax.experimental import pallas as pl
from jax.experimental.pallas import tpu as pltpu
from jax.experimental.pallas import tpu_sc as plsc

_VOCAB = 1000000
_D = 32
_BATCH = 4096
_HIST = 200
_N = _BATCH * _HIST
_NC, _NS = 2, 16
_NW = _NC * _NS
_BTG = 4
_CB = 128 * _BTG
_NBLK = (_HIST * (_BATCH // 128)) // _BTG
_PER_W = _NBLK // _NW
_BPH = _BATCH // _CB


def _make_gather():
  mesh = plsc.VectorSubcoreMesh(
      core_axis_name="c", subcore_axis_name="s",
      num_cores=_NC, num_subcores=_NS)

  @functools.partial(
      pl.kernel,
      mesh=mesh,
      out_type=jax.ShapeDtypeStruct((_HIST, 4, _BATCH * 8), jnp.float32),
      scratch_types=[
          pltpu.VMEM((2, _CB), jnp.int32),
          pltpu.VMEM((2, _CB, _D), jnp.float32),
          pltpu.VMEM((2, _CB * _D), jnp.float32),
          [pltpu.SemaphoreType.DMA] * 2,
          [pltpu.SemaphoreType.DMA] * 2,
          [pltpu.SemaphoreType.DMA] * 2,
      ],
      compiler_params=pltpu.CompilerParams(
          use_tc_tiling_on_sc=False, needs_layout_passes=False),
  )
  def gather(doc_hbm, tab_hbm, out_hbm, idx_v, rows_v, trans_v, isems,
             gsems, wsems):
    wid = lax.axis_index("s") * _NC + lax.axis_index("c")
    base_blk = wid * _PER_W

    lanes = lax.iota(jnp.int32, 16)

    def blk_coords(k):
      bid = base_blk + k
      return bid // _BPH, (bid % _BPH) * _CB

    def start_idx(k, s):
      h, off = blk_coords(k)
      pltpu.async_copy(doc_hbm.at[h, pl.ds(off, _CB)], idx_v.at[s],
                       isems[s])

    def wait_idx(s):
      pltpu.make_async_copy(doc_hbm.at[0, pl.ds(0, _CB)], idx_v.at[s],
                            isems[s]).wait()

    def start_gather(s):
      pltpu.async_copy(tab_hbm.at[idx_v.at[s]], rows_v.at[s], gsems[s])

    def wait_gather(s):
      pltpu.make_async_copy(tab_hbm.at[idx_v.at[s]], rows_v.at[s],
                            gsems[s]).wait()

    def start_writes(k, s):
      h, off = blk_coords(k)
      for ft in range(4):
        pltpu.async_copy(
            trans_v.at[s, pl.ds(ft * _CB * 8, _CB * 8)],
            out_hbm.at[h, ft, pl.ds(off * 8, _CB * 8)], wsems[s])

    def wait_writes(s):
      for ft in range(4):
        pltpu.make_async_copy(
            trans_v.at[s, pl.ds(ft * _CB * 8, _CB * 8)],
            out_hbm.at[0, 0, pl.ds(0, _CB * 8)], wsems[s]).wait()

    def transpose(s):
      def tbody(g16, carry):
        ridx = g16 * 16 + lanes
        goff = (g16 >> 3) * 1024 + (g16 & 7) * 16
        for f in range(32):
          vec = plsc.load_gather(
              rows_v.at[s], [ridx, jnp.full((16,), f, jnp.int32)])
          trans_v[s, pl.ds(goff + (f // 8) * (_CB * 8) + (f % 8) * 128,
                           16)] = vec
        return carry
      lax.fori_loop(0, _CB // 16, tbody, 0)

    start_idx(0, 0)
    start_idx(1, 1)
    wait_idx(0)
    start_gather(0)

    def body(g, carry):
      for s in range(2):
        k = 2 * g + s
        wait_gather(s)

        @pl.when(g < (_PER_W - 2) // 2 if s else g < (_PER_W - 1) // 2)
        def _():
          start_idx(k + 2, s)

        if s == 0:
          wait_idx(1)
          start_gather(1)
        else:
          @pl.when(g < _PER_W // 2 - 1)
          def _():
            wait_idx(0)
            start_gather(0)

        @pl.when(g >= 1)
        def _():
          wait_writes(s)

        transpose(s)
        start_writes(k, s)
      return carry

    lax.fori_loop(0, _PER_W // 2, body, 0)
    wait_writes(0)
    wait_writes(1)

  return gather


_gather = _make_gather()


def kernel(doc, table):
  doc_t = doc.T
  tab_flat = lax.optimization_barrier(table.reshape(_VOCAB * _D))
  tab2 = tab_flat.reshape(_VOCAB, _D)
  out5 = _gather(doc_t, tab2)
  out6 = out5.reshape(_HIST, 4, _BATCH // 128, 8, 128)
  return out6.transpose(2, 4, 0, 1, 3).reshape(_BATCH, _HIST, _D)


# speedup vs baseline: 1.3042x; 1.3042x over previous
import functools

import jax
import jax.numpy as jnp
from jax import lax
from jax.experimental import pallas as pl
from jax.experimental.pallas import tpu as pltpu
from jax.experimental.pallas import tpu_sc as plsc

_VOCAB = 1000000
_D = 32
_BATCH = 4096
_HIST = 200
_N = _BATCH * _HIST
_NC, _NS = 2, 16
_NW = _NC * _NS
_BTG = 4
_CB = 128 * _BTG
_NBLK = (_HIST * (_BATCH // 128)) // _BTG
_PER_W = _NBLK // _NW
_BPH = _BATCH // _CB


def _make_gather():
  mesh = plsc.VectorSubcoreMesh(
      core_axis_name="c", subcore_axis_name="s",
      num_cores=_NC, num_subcores=_NS)

  @functools.partial(
      pl.kernel,
      mesh=mesh,
      out_type=jax.ShapeDtypeStruct((_HIST, 4, _BATCH * 8), jnp.float32),
      scratch_types=[
          pltpu.VMEM((2, _CB), jnp.int32),
          pltpu.VMEM((2, _CB, _D), jnp.float32),
          pltpu.VMEM((2, _CB * _D), jnp.float32),
          [pltpu.SemaphoreType.DMA] * 2,
          [pltpu.SemaphoreType.DMA] * 2,
          [pltpu.SemaphoreType.DMA] * 2,
      ],
      compiler_params=pltpu.CompilerParams(
          use_tc_tiling_on_sc=False, needs_layout_passes=False),
  )
  def gather(doc_hbm, tab_hbm, out_hbm, idx_v, rows_v, trans_v, isems,
             gsems, wsems):
    wid = lax.axis_index("s") * _NC + lax.axis_index("c")
    base_blk = wid * _PER_W

    lanes = lax.iota(jnp.int32, 16)
    base0 = jnp.where(lanes < 8, lanes * 128,
                      _CB * 8 + (lanes - 8) * 128)

    def blk_coords(k):
      bid = base_blk + k
      return bid // _BPH, (bid % _BPH) * _CB

    def start_idx(k, s):
      h, off = blk_coords(k)
      pltpu.async_copy(doc_hbm.at[h, pl.ds(off, _CB)], idx_v.at[s],
                       isems[s])

    def wait_idx(s):
      pltpu.make_async_copy(doc_hbm.at[0, pl.ds(0, _CB)], idx_v.at[s],
                            isems[s]).wait()

    def start_gather(s):
      pltpu.async_copy(tab_hbm.at[idx_v.at[s]], rows_v.at[s], gsems[s])

    def wait_gather(s):
      pltpu.make_async_copy(tab_hbm.at[idx_v.at[s]], rows_v.at[s],
                            gsems[s]).wait()

    def start_writes(k, s):
      h, off = blk_coords(k)
      for ft in range(4):
        pltpu.async_copy(
            trans_v.at[s, pl.ds(ft * _CB * 8, _CB * 8)],
            out_hbm.at[h, ft, pl.ds(off * 8, _CB * 8)], wsems[s])

    def wait_writes(s):
      for ft in range(4):
        pltpu.make_async_copy(
            trans_v.at[s, pl.ds(ft * _CB * 8, _CB * 8)],
            out_hbm.at[0, 0, pl.ds(0, _CB * 8)], wsems[s]).wait()

    def transpose(s):
      @plsc.parallel_loop(0, _CB, unroll=8)
      def _(r):
        ilo = base0 + ((r >> 7) * 1024 + (r & 127))
        plsc.store_scatter(trans_v.at[s], [ilo],
                           rows_v[s, r, pl.ds(0, 16)])
        plsc.store_scatter(trans_v.at[s], [ilo + _CB * 16],
                           rows_v[s, r, pl.ds(16, 16)])

    start_idx(0, 0)
    start_idx(1, 1)
    wait_idx(0)
    start_gather(0)

    def body(g, carry):
      for s in range(2):
        k = 2 * g + s
        wait_gather(s)

        @pl.when(g < (_PER_W - 2) // 2 if s else g < (_PER_W - 1) // 2)
        def _():
          start_idx(k + 2, s)

        if s == 0:
          wait_idx(1)
          start_gather(1)
        else:
          @pl.when(g < _PER_W // 2 - 1)
          def _():
            wait_idx(0)
            start_gather(0)

        @pl.when(g >= 1)
        def _():
          wait_writes(s)

        transpose(s)
        start_writes(k, s)
      return carry

    lax.fori_loop(0, _PER_W // 2, body, 0)
    wait_writes(0)
    wait_writes(1)

  return gather


_gather = _make_gather()


def kernel(doc, table):
  doc_t = doc.T
  tab_flat = lax.optimization_barrier(table.reshape(_VOCAB * _D))
  tab2 = tab_flat.reshape(_VOCAB, _D)
  out5 = _gather(doc_t, tab2)
  out6 = out5.reshape(_HIST, 4, _BATCH // 128, 8, 128)
  return out6.transpose(2, 4, 0, 1, 3).reshape(_BATCH, _HIST, _D)


# speedup vs baseline: 1.3043x; 1.0001x over previous
import functools

import jax
import jax.numpy as jnp
from jax import lax
from jax.experimental import pallas as pl
from jax.experimental.pallas import tpu as pltpu
from jax.experimental.pallas import tpu_sc as plsc

_VOCAB = 1000000
_D = 32
_BATCH = 4096
_HIST = 200
_N = _BATCH * _HIST
_NC, _NS = 2, 16
_NW = _NC * _NS
_BTG = 4
_CB = 128 * _BTG
_NBLK = (_HIST * (_BATCH // 128)) // _BTG
_PER_W = _NBLK // _NW
_BPH = _BATCH // _CB


def _make_gather():
  mesh = plsc.VectorSubcoreMesh(
      core_axis_name="c", subcore_axis_name="s",
      num_cores=_NC, num_subcores=_NS)

  @functools.partial(
      pl.kernel,
      mesh=mesh,
      out_type=jax.ShapeDtypeStruct((_HIST, 4, _BATCH * 8), jnp.float32),
      scratch_types=[
          pltpu.VMEM((2, _CB), jnp.int32),
          pltpu.VMEM((2, _CB, _D), jnp.float32),
          pltpu.VMEM((2, _CB * _D), jnp.float32),
          [pltpu.SemaphoreType.DMA] * 2,
          [pltpu.SemaphoreType.DMA] * 2,
          [pltpu.SemaphoreType.DMA] * 2,
      ],
      compiler_params=pltpu.CompilerParams(
          use_tc_tiling_on_sc=False, needs_layout_passes=False),
  )
  def gather(doc_hbm, tab_hbm, out_hbm, idx_v, rows_v, trans_v, isems,
             gsems, wsems):
    wid = lax.axis_index("s") * _NC + lax.axis_index("c")
    base_blk = wid * _PER_W

    lanes = lax.iota(jnp.int32, 16)
    base0 = jnp.where(lanes < 8, lanes * 128,
                      _CB * 8 + (lanes - 8) * 128)

    def blk_coords(k):
      bid = base_blk + k
      return bid // _BPH, (bid % _BPH) * _CB

    def start_idx(k, s):
      h, off = blk_coords(k)
      pltpu.async_copy(doc_hbm.at[h, pl.ds(off, _CB)], idx_v.at[s],
                       isems[s])

    def wait_idx(s):
      pltpu.make_async_copy(doc_hbm.at[0, pl.ds(0, _CB)], idx_v.at[s],
                            isems[s]).wait()

    def start_gather(s):
      pltpu.async_copy(tab_hbm.at[idx_v.at[s]], rows_v.at[s], gsems[s])

    def wait_gather(s):
      pltpu.make_async_copy(tab_hbm.at[idx_v.at[s]], rows_v.at[s],
                            gsems[s]).wait()

    def start_writes(k, s):
      h, off = blk_coords(k)
      for ft in range(4):
        pltpu.async_copy(
            trans_v.at[s, pl.ds(ft * _CB * 8, _CB * 8)],
            out_hbm.at[h, ft, pl.ds(off * 8, _CB * 8)], wsems[s])

    def wait_writes(s):
      for ft in range(4):
        pltpu.make_async_copy(
            trans_v.at[s, pl.ds(ft * _CB * 8, _CB * 8)],
            out_hbm.at[0, 0, pl.ds(0, _CB * 8)], wsems[s]).wait()

    def transpose(s):
      @plsc.parallel_loop(0, _CB, unroll=16)
      def _(r):
        ilo = base0 + ((r >> 7) * 1024 + (r & 127))
        plsc.store_scatter(trans_v.at[s], [ilo],
                           rows_v[s, r, pl.ds(0, 16)])
        plsc.store_scatter(trans_v.at[s], [ilo + _CB * 16],
                           rows_v[s, r, pl.ds(16, 16)])

    start_idx(0, 0)
    start_idx(1, 1)
    wait_idx(0)
    start_gather(0)

    def body(g, carry):
      for s in range(2):
        k = 2 * g + s
        wait_gather(s)

        @pl.when(g < (_PER_W - 2) // 2 if s else g < (_PER_W - 1) // 2)
        def _():
          start_idx(k + 2, s)

        if s == 0:
          wait_idx(1)
          start_gather(1)
        else:
          @pl.when(g < _PER_W // 2 - 1)
          def _():
            wait_idx(0)
            start_gather(0)

        @pl.when(g >= 1)
        def _():
          wait_writes(s)

        transpose(s)
        start_writes(k, s)
      return carry

    lax.fori_loop(0, _PER_W // 2, body, 0)
    wait_writes(0)
    wait_writes(1)

  return gather


_gather = _make_gather()


def kernel(doc, table):
  doc_t = doc.T
  tab_flat = lax.optimization_barrier(table.reshape(_VOCAB * _D))
  tab2 = tab_flat.reshape(_VOCAB, _D)
  out5 = _gather(doc_t, tab2)
  out6 = out5.reshape(_HIST, 4, _BATCH // 128, 8, 128)
  return out6.transpose(2, 4, 0, 1, 3).reshape(_BATCH, _HIST, _D)
